# Initial kernel scaffold; baseline (speedup 1.0000x reference)
import jax
import jax.numpy as jnp
from jax.experimental import pallas as pl


def kernel(x, users_emb, items_emb, brands_emb, W1, b1, W2, b2, W3, b3):
    raise NotImplementedError("write your pallas kernel here")


# speedup vs baseline: 4.9416x; 4.9416x over previous
import functools

import jax
import jax.numpy as jnp
from jax import lax
from jax.experimental import pallas as pl
from jax.experimental.pallas import tpu as pltpu
from jax.experimental.pallas import tpu_sc as plsc

DIM = 128
BRAND_DIM = 16
BATCH = 16384

NC = 2
NS = 16
NW = NC * NS
BPW = BATCH // NW
CHUNK = 128
K = BPW // CHUNK

@functools.cache
def _make_sc_gather():
    mesh = plsc.VectorSubcoreMesh(core_axis_name="c", subcore_axis_name="s")

    @functools.partial(
        pl.kernel,
        mesh=mesh,
        out_type=(
            jax.ShapeDtypeStruct((BATCH, DIM), jnp.float32),
            jax.ShapeDtypeStruct((BATCH, DIM), jnp.float32),
            jax.ShapeDtypeStruct((BATCH, DIM), jnp.float32),
        ),
        scratch_types=[
            pltpu.VMEM((K, CHUNK), jnp.int32),
            pltpu.VMEM((K, CHUNK), jnp.int32),
            pltpu.VMEM((K, CHUNK), jnp.int32),
            pltpu.VMEM((BPW, DIM), jnp.float32),
            pltpu.SemaphoreType.DMA,
        ],
    )
    def _sc_gather(uidx_hbm, iidx_hbm, bidx_hbm, users_hbm, items_hbm,
                   brands_hbm, uout, iout, bout, uidx_v, iidx_v, bidx_v,
                   rows_v, sem):
        wid = lax.axis_index("s") * NC + lax.axis_index("c")
        base = wid * BPW
        pltpu.sync_copy(uidx_hbm.at[wid], uidx_v)
        pltpu.sync_copy(iidx_hbm.at[wid], iidx_v)
        pltpu.sync_copy(bidx_hbm.at[wid], bidx_v)

        for idx_v, table, out in ((uidx_v, users_hbm, uout),
                                  (iidx_v, items_hbm, iout),
                                  (bidx_v, brands_hbm, bout)):
            copies = [
                pltpu.async_copy(table.at[idx_v.at[j]],
                                 rows_v.at[pl.ds(j * CHUNK, CHUNK)], sem)
                for j in range(K)
            ]
            for c in copies:
                c.wait()
            pltpu.sync_copy(rows_v, out.at[pl.ds(base, BPW)])

    return _sc_gather


BM = 2048


def _mlp_body(u, it, br, xp, w1u, w1i, w1b, w1x, b1, w2, b2, w3, b3, o):
    h = jnp.dot(u[:], w1u[:], preferred_element_type=jnp.float32)
    h = h + jnp.dot(it[:], w1i[:], preferred_element_type=jnp.float32)
    h = h + jnp.dot(br[:], w1b[:], preferred_element_type=jnp.float32)
    h = h + jnp.dot(xp[:], w1x[:], preferred_element_type=jnp.float32)
    h = jnp.maximum(h + b1[:], 0.0)
    h2 = jnp.maximum(jnp.dot(h, w2[:], preferred_element_type=jnp.float32) + b2[:], 0.0)
    o[:] = jnp.tanh(jnp.dot(h2, w3[:], preferred_element_type=jnp.float32) + b3[:])


def _mlp(u, it, br, xp, w1u, w1i, w1b, w1x, b1, w2, b2, w3, b3):
    grid = (BATCH // BM,)
    row = lambda i: (i, 0)
    rep = lambda i: (0, 0)
    return pl.pallas_call(
        _mlp_body,
        grid=grid,
        in_specs=[
            pl.BlockSpec((BM, DIM), row),
            pl.BlockSpec((BM, DIM), row),
            pl.BlockSpec((BM, DIM), row),
            pl.BlockSpec((BM, 32), row),
            pl.BlockSpec((DIM, DIM), rep),
            pl.BlockSpec((DIM, DIM), rep),
            pl.BlockSpec((DIM, DIM), rep),
            pl.BlockSpec((32, DIM), rep),
            pl.BlockSpec((1, DIM), rep),
            pl.BlockSpec((DIM, 32), rep),
            pl.BlockSpec((1, 32), rep),
            pl.BlockSpec((32, 1), rep),
            pl.BlockSpec((1, 1), rep),
        ],
        out_specs=pl.BlockSpec((BM, 1), row),
        out_shape=jax.ShapeDtypeStruct((BATCH, 1), jnp.float32),
    )(u, it, br, xp, w1u, w1i, w1b, w1x, b1, w2, b2, w3, b3)


def kernel(x, users_emb, items_emb, brands_emb, W1, b1, W2, b2, W3, b3):
    uidx = x[:, 0].astype(jnp.int32).reshape(NW, K, CHUNK)
    iidx = x[:, 1].astype(jnp.int32).reshape(NW, K, CHUNK)
    bidx = x[:, 19].astype(jnp.int32).reshape(NW, K, CHUNK)

    brands_p = jnp.pad(brands_emb, ((0, 0), (0, DIM - BRAND_DIM)))
    u_g, i_g, b_g = _make_sc_gather()(uidx, iidx, bidx,
                                      users_emb, items_emb, brands_p)

    zeros2 = jnp.zeros((2, DIM), jnp.float32)
    W1x = jnp.concatenate(
        [zeros2, W1[272:288], zeros2, W1[288:299], jnp.zeros((1, DIM), jnp.float32)],
        axis=0)
    xp = jnp.pad(x, ((0, 0), (0, 1)))

    W1b = jnp.pad(W1[256:272], ((0, DIM - BRAND_DIM), (0, 0)))
    return _mlp(u_g, i_g, b_g, xp,
                W1[0:128], W1[128:256], W1b, W1x,
                b1.reshape(1, DIM), W2, b2.reshape(1, 32),
                W3, b3.reshape(1, 1))
